# Initial kernel scaffold; baseline (speedup 1.0000x reference)
import jax
import jax.numpy as jnp
from jax.experimental import pallas as pl


def kernel(embeddings, vocab_embeddings):
    raise NotImplementedError("write your pallas kernel here")


# speedup vs baseline: 1.1734x; 1.1734x over previous
import functools

import jax
import jax.numpy as jnp
from jax import lax
from jax.experimental import pallas as pl
from jax.experimental.pallas import tpu as pltpu
from jax.experimental.pallas import tpu_sc as plsc

EPS = 0.1
T_TILE = 1024
V_TILE = 2048

N_TOK = 16384
N_VOCAB = 8192
D = 64

_NC, _NS = 2, 16
_NW = _NC * _NS
_BPW = N_TOK // _NW


def _argmax_body(flat_ref, vocab_ref, ids_ref, run_max, run_idx):
    v = pl.program_id(1)
    nv = pl.num_programs(1)

    emb = flat_ref[...]
    en = jnp.sqrt(jnp.sum(emb * emb, axis=1, keepdims=True))
    emb_n = (emb / jnp.maximum(en, 1e-12)).astype(jnp.bfloat16)

    voc = vocab_ref[...]
    vn = jnp.sqrt(jnp.sum(voc * voc, axis=1, keepdims=True))
    voc_n = (voc / jnp.maximum(vn, 1e-12)).astype(jnp.bfloat16)

    sim = jax.lax.dot_general(
        emb_n, voc_n, (((1,), (1,)), ((), ())),
        preferred_element_type=jnp.float32,
    )
    m = jnp.max(sim, axis=1, keepdims=True)
    iota = jax.lax.broadcasted_iota(jnp.int32, sim.shape, 1) + v * V_TILE
    idx = jnp.min(jnp.where(sim == m, iota, jnp.int32(2**31 - 1)),
                  axis=1, keepdims=True)

    @pl.when(v == 0)
    def _():
        run_max[...] = m
        run_idx[...] = idx

    @pl.when(v > 0)
    def _():
        better = m > run_max[...]
        run_idx[...] = jnp.where(better, idx, run_idx[...])
        run_max[...] = jnp.maximum(m, run_max[...])

    @pl.when(v == nv - 1)
    def _():
        ids_ref[...] = run_idx[...]


def _anchor_ids(flat, vocab):
    grid = (N_TOK // T_TILE, N_VOCAB // V_TILE)
    ids = pl.pallas_call(
        _argmax_body,
        grid=grid,
        in_specs=[
            pl.BlockSpec((T_TILE, D), lambda t, v: (t, 0)),
            pl.BlockSpec((V_TILE, D), lambda t, v: (v, 0)),
        ],
        out_specs=pl.BlockSpec((T_TILE, 1), lambda t, v: (t, 0)),
        out_shape=jax.ShapeDtypeStruct((N_TOK, 1), jnp.int32),
        scratch_shapes=[
            pltpu.VMEM((T_TILE, 1), jnp.float32),
            pltpu.VMEM((T_TILE, 1), jnp.int32),
        ],
    )(flat, vocab)
    return ids[:, 0]


def _gather_body(ids_hbm, vocab_hbm, out_hbm, idx_v, rows_v, sem):
    wid = lax.axis_index("s") * _NC + lax.axis_index("c")
    base = wid * _BPW
    pltpu.sync_copy(ids_hbm.at[pl.ds(base, _BPW)], idx_v)
    pltpu.async_copy(vocab_hbm.at[idx_v], rows_v, sem).wait()
    pltpu.sync_copy(rows_v, out_hbm.at[pl.ds(base, _BPW)])


_sc_gather = pl.kernel(
    _gather_body,
    out_type=jax.ShapeDtypeStruct((N_TOK, 128), jnp.float32),
    mesh=plsc.VectorSubcoreMesh(core_axis_name="c", subcore_axis_name="s"),
    scratch_types=[
        pltpu.VMEM((_BPW,), jnp.int32),
        pltpu.VMEM((_BPW, 128), jnp.float32),
        pltpu.SemaphoreType.DMA,
    ],
)


def _clip_body(flat_ref, anc_ref, out_ref):
    f = flat_ref[...]
    a = anc_ref[:, :D]
    off = f - a
    on2 = jnp.sum(off * off, axis=1, keepdims=True)
    an2 = jnp.sum(a * a, axis=1, keepdims=True)
    scale = jnp.minimum(EPS * jnp.sqrt(an2) / (jnp.sqrt(on2) + 1e-8), 1.0)
    out_ref[...] = a + off * scale


def _clip(flat, anchors):
    c_tile = 2048
    return pl.pallas_call(
        _clip_body,
        grid=(N_TOK // c_tile,),
        in_specs=[
            pl.BlockSpec((c_tile, D), lambda t: (t, 0)),
            pl.BlockSpec((c_tile, 128), lambda t: (t, 0)),
        ],
        out_specs=pl.BlockSpec((c_tile, D), lambda t: (t, 0)),
        out_shape=jax.ShapeDtypeStruct((N_TOK, D), jnp.float32),
    )(flat, anchors)


def kernel(embeddings, vocab_embeddings):
    B, S, _ = embeddings.shape
    flat = embeddings.reshape(-1, D)
    ids = _anchor_ids(flat, vocab_embeddings)
    vocab_pad = jnp.pad(vocab_embeddings, ((0, 0), (0, 128 - D)))
    anchors = _sc_gather(ids, vocab_pad)
    res = _clip(flat, anchors)
    return res.reshape(B, S, D), ids.reshape(B, S)


# speedup vs baseline: 1.3857x; 1.1810x over previous
import functools

import jax
import jax.numpy as jnp
from jax import lax
from jax.experimental import pallas as pl
from jax.experimental.pallas import tpu as pltpu
from jax.experimental.pallas import tpu_sc as plsc

EPS = 0.1
T_TILE = 1024
V_TILE = 2048

N_TOK = 16384
N_VOCAB = 8192
D = 64

_NC, _NS = 2, 16
_NW = _NC * _NS
_BPW = N_TOK // _NW


def _argmax_body(flat_ref, vocab_ref, ids_ref, run_max, run_idx,
                 emb_scr, voc_scr):
    t = pl.program_id(0)
    v = pl.program_id(1)
    nv = pl.num_programs(1)

    @pl.when(v == 0)
    def _():
        emb = flat_ref[...]
        en = jnp.sqrt(jnp.sum(emb * emb, axis=1, keepdims=True))
        emb_scr[...] = (emb / jnp.maximum(en, 1e-12)).astype(jnp.bfloat16)

    @pl.when(t == 0)
    def _():
        voc = vocab_ref[...]
        vn = jnp.sqrt(jnp.sum(voc * voc, axis=1, keepdims=True))
        voc_scr[v] = (voc / jnp.maximum(vn, 1e-12)).astype(jnp.bfloat16)

    sim = jax.lax.dot_general(
        emb_scr[...], voc_scr[v], (((1,), (1,)), ((), ())),
        preferred_element_type=jnp.float32,
    )
    m = jnp.max(sim, axis=1, keepdims=True)
    iota_f = jax.lax.broadcasted_iota(
        jnp.int32, (1, V_TILE), 1).astype(jnp.float32)
    idx_f = jnp.min(jnp.where(sim == m, iota_f, jnp.float32(jnp.inf)),
                    axis=1, keepdims=True)
    idx = idx_f.astype(jnp.int32) + v * V_TILE

    @pl.when(v == 0)
    def _():
        run_max[...] = m
        run_idx[...] = idx

    @pl.when(v > 0)
    def _():
        better = m > run_max[...]
        run_idx[...] = jnp.where(better, idx, run_idx[...])
        run_max[...] = jnp.maximum(m, run_max[...])

    @pl.when(v == nv - 1)
    def _():
        ids_ref[...] = run_idx[...]


def _anchor_ids(flat, vocab):
    grid = (N_TOK // T_TILE, N_VOCAB // V_TILE)
    ids = pl.pallas_call(
        _argmax_body,
        grid=grid,
        in_specs=[
            pl.BlockSpec((T_TILE, D), lambda t, v: (t, 0)),
            pl.BlockSpec((V_TILE, D), lambda t, v: (v, 0)),
        ],
        out_specs=pl.BlockSpec((T_TILE, 1), lambda t, v: (t, 0)),
        out_shape=jax.ShapeDtypeStruct((N_TOK, 1), jnp.int32),
        scratch_shapes=[
            pltpu.VMEM((T_TILE, 1), jnp.float32),
            pltpu.VMEM((T_TILE, 1), jnp.int32),
            pltpu.VMEM((T_TILE, D), jnp.bfloat16),
            pltpu.VMEM((N_VOCAB // V_TILE, V_TILE, D), jnp.bfloat16),
        ],
    )(flat, vocab)
    return ids[:, 0]


def _gather_body(ids_hbm, vocab_hbm, out_hbm, idx_v, rows_v, sem):
    wid = lax.axis_index("s") * _NC + lax.axis_index("c")
    base = wid * _BPW
    pltpu.sync_copy(ids_hbm.at[pl.ds(base, _BPW)], idx_v)
    pltpu.async_copy(vocab_hbm.at[idx_v], rows_v, sem).wait()
    pltpu.sync_copy(rows_v, out_hbm.at[pl.ds(base, _BPW)])


@functools.cache
def _sc_gather_kernel():
    return pl.kernel(
        _gather_body,
        out_type=jax.ShapeDtypeStruct((N_TOK, 128), jnp.float32),
        mesh=plsc.VectorSubcoreMesh(core_axis_name="c", subcore_axis_name="s"),
        scratch_types=[
            pltpu.VMEM((_BPW,), jnp.int32),
            pltpu.VMEM((_BPW, 128), jnp.float32),
            pltpu.SemaphoreType.DMA,
        ],
    )


def _clip_body(flat_ref, anc_ref, out_ref):
    f = flat_ref[...]
    a = anc_ref[:, :D]
    off = f - a
    on2 = jnp.sum(off * off, axis=1, keepdims=True)
    an2 = jnp.sum(a * a, axis=1, keepdims=True)
    scale = jnp.minimum(EPS * jnp.sqrt(an2) / (jnp.sqrt(on2) + 1e-8), 1.0)
    out_ref[...] = a + off * scale


def _clip(flat, anchors):
    c_tile = 2048
    return pl.pallas_call(
        _clip_body,
        grid=(N_TOK // c_tile,),
        in_specs=[
            pl.BlockSpec((c_tile, D), lambda t: (t, 0)),
            pl.BlockSpec((c_tile, 128), lambda t: (t, 0)),
        ],
        out_specs=pl.BlockSpec((c_tile, D), lambda t: (t, 0)),
        out_shape=jax.ShapeDtypeStruct((N_TOK, D), jnp.float32),
    )(flat, anchors)


def kernel(embeddings, vocab_embeddings):
    B, S, _ = embeddings.shape
    flat = embeddings.reshape(-1, D)
    ids = _anchor_ids(flat, vocab_embeddings)
    vocab_pad = jnp.pad(vocab_embeddings, ((0, 0), (0, 128 - D)))
    anchors = _sc_gather_kernel()(ids, vocab_pad)
    res = _clip(flat, anchors)
    return res.reshape(B, S, D), ids.reshape(B, S)


# speedup vs baseline: 1.4548x; 1.0499x over previous
import functools

import jax
import jax.numpy as jnp
from jax import lax
from jax.experimental import pallas as pl
from jax.experimental.pallas import tpu as pltpu
from jax.experimental.pallas import tpu_sc as plsc

EPS = 0.1
T_TILE = 2048
V_TILE = 2048

N_TOK = 16384
N_VOCAB = 8192
D = 64

_NC, _NS = 2, 16
_NW = _NC * _NS
_BPW = N_TOK // _NW


def _argmax_body(flat_ref, vocab_ref, ids_ref, run_max, run_idx,
                 emb_scr, voc_scr):
    t = pl.program_id(0)
    v = pl.program_id(1)
    nv = pl.num_programs(1)

    @pl.when(v == 0)
    def _():
        emb = flat_ref[...]
        en = jnp.sqrt(jnp.sum(emb * emb, axis=1, keepdims=True))
        emb_scr[...] = (emb / jnp.maximum(en, 1e-12)).astype(jnp.bfloat16)

    @pl.when(t == 0)
    def _():
        voc = vocab_ref[...]
        vn = jnp.sqrt(jnp.sum(voc * voc, axis=1, keepdims=True))
        voc_scr[v] = (voc / jnp.maximum(vn, 1e-12)).astype(jnp.bfloat16)

    sim = jax.lax.dot_general(
        emb_scr[...], voc_scr[v], (((1,), (1,)), ((), ())),
        preferred_element_type=jnp.float32,
    )
    m = jnp.max(sim, axis=1, keepdims=True)
    iota_f = jax.lax.broadcasted_iota(
        jnp.int32, (1, V_TILE), 1).astype(jnp.float32)
    idx_f = jnp.min(jnp.where(sim == m, iota_f, jnp.float32(jnp.inf)),
                    axis=1, keepdims=True)
    idx = idx_f.astype(jnp.int32) + v * V_TILE

    @pl.when(v == 0)
    def _():
        run_max[...] = m
        run_idx[...] = idx

    @pl.when(v > 0)
    def _():
        better = m > run_max[...]
        run_idx[...] = jnp.where(better, idx, run_idx[...])
        run_max[...] = jnp.maximum(m, run_max[...])

    @pl.when(v == nv - 1)
    def _():
        ids_ref[...] = run_idx[...]


def _anchor_ids(flat, vocab):
    grid = (N_TOK // T_TILE, N_VOCAB // V_TILE)
    ids = pl.pallas_call(
        _argmax_body,
        grid=grid,
        in_specs=[
            pl.BlockSpec((T_TILE, D), lambda t, v: (t, 0)),
            pl.BlockSpec((V_TILE, D), lambda t, v: (v, 0)),
        ],
        out_specs=pl.BlockSpec((T_TILE, 1), lambda t, v: (t, 0)),
        out_shape=jax.ShapeDtypeStruct((N_TOK, 1), jnp.int32),
        scratch_shapes=[
            pltpu.VMEM((T_TILE, 1), jnp.float32),
            pltpu.VMEM((T_TILE, 1), jnp.int32),
            pltpu.VMEM((T_TILE, D), jnp.bfloat16),
            pltpu.VMEM((N_VOCAB // V_TILE, V_TILE, D), jnp.bfloat16),
        ],
    )(flat, vocab)
    return ids[:, 0]


def _gather_body(ids_hbm, vocab_hbm, out_hbm, idx_v, rows_v, sem):
    wid = lax.axis_index("s") * _NC + lax.axis_index("c")
    base = wid * _BPW
    pltpu.sync_copy(ids_hbm.at[pl.ds(base, _BPW)], idx_v)
    pltpu.async_copy(vocab_hbm.at[idx_v], rows_v, sem).wait()
    pltpu.sync_copy(rows_v, out_hbm.at[pl.ds(base, _BPW)])


@functools.cache
def _sc_gather_kernel():
    return pl.kernel(
        _gather_body,
        out_type=jax.ShapeDtypeStruct((N_TOK, 128), jnp.float32),
        mesh=plsc.VectorSubcoreMesh(core_axis_name="c", subcore_axis_name="s"),
        scratch_types=[
            pltpu.VMEM((_BPW,), jnp.int32),
            pltpu.VMEM((_BPW, 128), jnp.float32),
            pltpu.SemaphoreType.DMA,
        ],
    )


def _clip_body(flat_ref, anc_ref, out_ref):
    f = flat_ref[...]
    a = anc_ref[:, :D]
    off = f - a
    on2 = jnp.sum(off * off, axis=1, keepdims=True)
    an2 = jnp.sum(a * a, axis=1, keepdims=True)
    scale = jnp.minimum(EPS * jnp.sqrt(an2) / (jnp.sqrt(on2) + 1e-8), 1.0)
    out_ref[...] = a + off * scale


def _clip(flat, anchors):
    c_tile = 2048
    return pl.pallas_call(
        _clip_body,
        grid=(N_TOK // c_tile,),
        in_specs=[
            pl.BlockSpec((c_tile, D), lambda t: (t, 0)),
            pl.BlockSpec((c_tile, 128), lambda t: (t, 0)),
        ],
        out_specs=pl.BlockSpec((c_tile, D), lambda t: (t, 0)),
        out_shape=jax.ShapeDtypeStruct((N_TOK, D), jnp.float32),
    )(flat, anchors)


def kernel(embeddings, vocab_embeddings):
    B, S, _ = embeddings.shape
    flat = embeddings.reshape(-1, D)
    ids = _anchor_ids(flat, vocab_embeddings)
    vocab_pad = jnp.pad(vocab_embeddings, ((0, 0), (0, 128 - D)))
    anchors = _sc_gather_kernel()(ids, vocab_pad)
    res = _clip(flat, anchors)
    return res.reshape(B, S, D), ids.reshape(B, S)


# speedup vs baseline: 1.5785x; 1.0850x over previous
import functools

import jax
import jax.numpy as jnp
from jax import lax
from jax.experimental import pallas as pl
from jax.experimental.pallas import tpu as pltpu
from jax.experimental.pallas import tpu_sc as plsc

EPS = 0.1
T_TILE = 1024
V_TILE = 2048

N_TOK = 16384
N_VOCAB = 8192
D = 64

_NC, _NS = 2, 16
_NW = _NC * _NS
_BPW = N_TOK // _NW


NT = N_TOK // T_TILE
NV = N_VOCAB // V_TILE


def _argmax_body(flat_ref, vocab_ref, ids_ref, voc_scr):
    t = pl.program_id(0)

    @pl.when(t == 0)
    def _():
        voc = vocab_ref[...]
        vn = jnp.sqrt(jnp.sum(voc * voc, axis=1, keepdims=True))
        voc_scr[...] = (voc / jnp.maximum(vn, 1e-12)).astype(jnp.bfloat16)

    emb = flat_ref[...]
    en = jnp.sqrt(jnp.sum(emb * emb, axis=1, keepdims=True))
    emb_n = (emb / jnp.maximum(en, 1e-12)).astype(jnp.bfloat16)

    iota_f = jax.lax.broadcasted_iota(
        jnp.int32, (1, V_TILE), 1).astype(jnp.float32)

    ri = None
    rm = None
    for v in range(NV):
        sim = jax.lax.dot_general(
            emb_n, voc_scr[v * V_TILE:(v + 1) * V_TILE, :],
            (((1,), (1,)), ((), ())),
            preferred_element_type=jnp.float32,
        )
        m = jnp.max(sim, axis=1, keepdims=True)
        idx_f = jnp.min(jnp.where(sim == m, iota_f, jnp.float32(jnp.inf)),
                        axis=1, keepdims=True)
        idx = idx_f.astype(jnp.int32) + v * V_TILE
        if v == 0:
            rm, ri = m, idx
        else:
            better = m > rm
            ri = jnp.where(better, idx, ri)
            rm = jnp.maximum(m, rm)
    ids_ref[...] = ri


def _anchor_ids(flat, vocab):
    ids = pl.pallas_call(
        _argmax_body,
        grid=(NT,),
        in_specs=[
            pl.BlockSpec((T_TILE, D), lambda t: (t, 0)),
            pl.BlockSpec((N_VOCAB, D), lambda t: (0, 0)),
        ],
        out_specs=pl.BlockSpec((T_TILE, 1), lambda t: (t, 0)),
        out_shape=jax.ShapeDtypeStruct((N_TOK, 1), jnp.int32),
        scratch_shapes=[
            pltpu.VMEM((N_VOCAB, D), jnp.bfloat16),
        ],
    )(flat, vocab)
    return ids[:, 0]


def _gather_body(ids_hbm, vocab_hbm, out_hbm, idx_v, rows_v, sem):
    wid = lax.axis_index("s") * _NC + lax.axis_index("c")
    base = wid * _BPW
    pltpu.sync_copy(ids_hbm.at[pl.ds(base, _BPW)], idx_v)
    pltpu.async_copy(vocab_hbm.at[idx_v], rows_v, sem).wait()
    pltpu.sync_copy(rows_v, out_hbm.at[pl.ds(base, _BPW)])


@functools.cache
def _sc_gather_kernel():
    return pl.kernel(
        _gather_body,
        out_type=jax.ShapeDtypeStruct((N_TOK, 128), jnp.float32),
        mesh=plsc.VectorSubcoreMesh(core_axis_name="c", subcore_axis_name="s"),
        scratch_types=[
            pltpu.VMEM((_BPW,), jnp.int32),
            pltpu.VMEM((_BPW, 128), jnp.float32),
            pltpu.SemaphoreType.DMA,
        ],
    )


def _clip_body(flat_ref, anc_ref, out_ref):
    f = flat_ref[...]
    a = anc_ref[:, :D]
    off = f - a
    on2 = jnp.sum(off * off, axis=1, keepdims=True)
    an2 = jnp.sum(a * a, axis=1, keepdims=True)
    scale = jnp.minimum(EPS * jnp.sqrt(an2) / (jnp.sqrt(on2) + 1e-8), 1.0)
    out_ref[...] = a + off * scale


def _clip(flat, anchors):
    c_tile = 2048
    return pl.pallas_call(
        _clip_body,
        grid=(N_TOK // c_tile,),
        in_specs=[
            pl.BlockSpec((c_tile, D), lambda t: (t, 0)),
            pl.BlockSpec((c_tile, 128), lambda t: (t, 0)),
        ],
        out_specs=pl.BlockSpec((c_tile, D), lambda t: (t, 0)),
        out_shape=jax.ShapeDtypeStruct((N_TOK, D), jnp.float32),
    )(flat, anchors)


def kernel(embeddings, vocab_embeddings):
    B, S, _ = embeddings.shape
    flat = embeddings.reshape(-1, D)
    ids = _anchor_ids(flat, vocab_embeddings)
    vocab_pad = jnp.pad(vocab_embeddings, ((0, 0), (0, 128 - D)))
    anchors = _sc_gather_kernel()(ids, vocab_pad)
    res = _clip(flat, anchors)
    return res.reshape(B, S, D), ids.reshape(B, S)


# speedup vs baseline: 2.0359x; 1.2897x over previous
import functools

import jax
import jax.numpy as jnp
from jax import lax
from jax.experimental import pallas as pl
from jax.experimental.pallas import tpu as pltpu
from jax.experimental.pallas import tpu_sc as plsc

EPS = 0.1
T_TILE = 1024
V_TILE = 2048

N_TOK = 16384
N_VOCAB = 8192
D = 64

_NC, _NS = 2, 16
_NW = _NC * _NS
_BPW = N_TOK // _NW


NT = N_TOK // T_TILE
NV = N_VOCAB // V_TILE


def _argmax_body(flat_ref, vocab_ref, ids_ref, voc_scr):
    t = pl.program_id(0)

    @pl.when(t == 0)
    def _():
        voc = vocab_ref[...]
        vn = jnp.sqrt(jnp.sum(voc * voc, axis=1, keepdims=True))
        voc_scr[...] = (voc / jnp.maximum(vn, 1e-12)).astype(jnp.bfloat16)

    emb = flat_ref[...]
    en = jnp.sqrt(jnp.sum(emb * emb, axis=1, keepdims=True))
    emb_n = (emb / jnp.maximum(en, 1e-12)).astype(jnp.bfloat16)

    rm = jnp.full((T_TILE, 128), -jnp.inf, jnp.float32)
    ri = jnp.zeros((T_TILE, 128), jnp.float32)
    for v in range(NV):
        sim = jax.lax.dot_general(
            emb_n, voc_scr[v * V_TILE:(v + 1) * V_TILE, :],
            (((1,), (1,)), ((), ())),
            preferred_element_type=jnp.float32,
        )
        for c in range(V_TILE // 128):
            chunk = sim[:, c * 128:(c + 1) * 128]
            gt = chunk > rm
            ri = jnp.where(gt, jnp.float32(v * (V_TILE // 128) + c), ri)
            rm = jnp.where(gt, chunk, rm)

    m = jnp.max(rm, axis=1, keepdims=True)
    lane_f = jax.lax.broadcasted_iota(
        jnp.int32, (1, 128), 1).astype(jnp.float32)
    enc = ri * jnp.float32(128.0) + lane_f
    idx_f = jnp.min(jnp.where(rm == m, enc, jnp.float32(jnp.inf)),
                    axis=1, keepdims=True)
    ids_ref[...] = idx_f.astype(jnp.int32)


def _anchor_ids(flat, vocab):
    ids = pl.pallas_call(
        _argmax_body,
        grid=(NT,),
        in_specs=[
            pl.BlockSpec((T_TILE, D), lambda t: (t, 0)),
            pl.BlockSpec((N_VOCAB, D), lambda t: (0, 0)),
        ],
        out_specs=pl.BlockSpec((T_TILE, 1), lambda t: (t, 0)),
        out_shape=jax.ShapeDtypeStruct((N_TOK, 1), jnp.int32),
        scratch_shapes=[
            pltpu.VMEM((N_VOCAB, D), jnp.bfloat16),
        ],
    )(flat, vocab)
    return ids[:, 0]


def _gather_body(ids_hbm, vocab_hbm, out_hbm, idx_v, rows_v, sem):
    wid = lax.axis_index("s") * _NC + lax.axis_index("c")
    base = wid * _BPW
    pltpu.sync_copy(ids_hbm.at[pl.ds(base, _BPW)], idx_v)
    pltpu.async_copy(vocab_hbm.at[idx_v], rows_v, sem).wait()
    pltpu.sync_copy(rows_v, out_hbm.at[pl.ds(base, _BPW)])


@functools.cache
def _sc_gather_kernel():
    return pl.kernel(
        _gather_body,
        out_type=jax.ShapeDtypeStruct((N_TOK, 128), jnp.float32),
        mesh=plsc.VectorSubcoreMesh(core_axis_name="c", subcore_axis_name="s"),
        scratch_types=[
            pltpu.VMEM((_BPW,), jnp.int32),
            pltpu.VMEM((_BPW, 128), jnp.float32),
            pltpu.SemaphoreType.DMA,
        ],
    )


def _clip_body(flat_ref, anc_ref, out_ref):
    f = flat_ref[...]
    a = anc_ref[:, :D]
    off = f - a
    on2 = jnp.sum(off * off, axis=1, keepdims=True)
    an2 = jnp.sum(a * a, axis=1, keepdims=True)
    scale = jnp.minimum(EPS * jnp.sqrt(an2) / (jnp.sqrt(on2) + 1e-8), 1.0)
    out_ref[...] = a + off * scale


def _clip(flat, anchors):
    c_tile = 2048
    return pl.pallas_call(
        _clip_body,
        grid=(N_TOK // c_tile,),
        in_specs=[
            pl.BlockSpec((c_tile, D), lambda t: (t, 0)),
            pl.BlockSpec((c_tile, 128), lambda t: (t, 0)),
        ],
        out_specs=pl.BlockSpec((c_tile, D), lambda t: (t, 0)),
        out_shape=jax.ShapeDtypeStruct((N_TOK, D), jnp.float32),
    )(flat, anchors)


def kernel(embeddings, vocab_embeddings):
    B, S, _ = embeddings.shape
    flat = embeddings.reshape(-1, D)
    ids = _anchor_ids(flat, vocab_embeddings)
    vocab_pad = jnp.pad(vocab_embeddings, ((0, 0), (0, 128 - D)))
    anchors = _sc_gather_kernel()(ids, vocab_pad)
    res = _clip(flat, anchors)
    return res.reshape(B, S, D), ids.reshape(B, S)


# speedup vs baseline: 2.0600x; 1.0119x over previous
import functools

import jax
import jax.numpy as jnp
from jax import lax
from jax.experimental import pallas as pl
from jax.experimental.pallas import tpu as pltpu
from jax.experimental.pallas import tpu_sc as plsc

EPS = 0.1
T_TILE = 2048
V_TILE = 2048

N_TOK = 16384
N_VOCAB = 8192
D = 64

_NC, _NS = 2, 16
_NW = _NC * _NS
_BPW = N_TOK // _NW


NT = N_TOK // T_TILE
NV = N_VOCAB // V_TILE


def _argmax_body(flat_ref, vocab_ref, ids_ref, voc_scr):
    t = pl.program_id(0)

    @pl.when(t == 0)
    def _():
        voc = vocab_ref[...]
        vn = jnp.sqrt(jnp.sum(voc * voc, axis=1, keepdims=True))
        voc_scr[...] = (voc / jnp.maximum(vn, 1e-12)).astype(jnp.bfloat16)

    emb = flat_ref[...]
    en = jnp.sqrt(jnp.sum(emb * emb, axis=1, keepdims=True))
    emb_n = (emb / jnp.maximum(en, 1e-12)).astype(jnp.bfloat16)

    rm = jnp.full((T_TILE, 128), -jnp.inf, jnp.float32)
    ri = jnp.zeros((T_TILE, 128), jnp.float32)
    for v in range(NV):
        sim = jax.lax.dot_general(
            emb_n, voc_scr[v * V_TILE:(v + 1) * V_TILE, :],
            (((1,), (1,)), ((), ())),
            preferred_element_type=jnp.float32,
        )
        for c in range(V_TILE // 128):
            chunk = sim[:, c * 128:(c + 1) * 128]
            gt = chunk > rm
            ri = jnp.where(gt, jnp.float32(v * (V_TILE // 128) + c), ri)
            rm = jnp.where(gt, chunk, rm)

    m = jnp.max(rm, axis=1, keepdims=True)
    lane_f = jax.lax.broadcasted_iota(
        jnp.int32, (1, 128), 1).astype(jnp.float32)
    enc = ri * jnp.float32(128.0) + lane_f
    idx_f = jnp.min(jnp.where(rm == m, enc, jnp.float32(jnp.inf)),
                    axis=1, keepdims=True)
    ids_ref[...] = idx_f.astype(jnp.int32)


def _anchor_ids(flat, vocab):
    ids = pl.pallas_call(
        _argmax_body,
        grid=(NT,),
        in_specs=[
            pl.BlockSpec((T_TILE, D), lambda t: (t, 0)),
            pl.BlockSpec((N_VOCAB, D), lambda t: (0, 0)),
        ],
        out_specs=pl.BlockSpec((T_TILE, 1), lambda t: (t, 0)),
        out_shape=jax.ShapeDtypeStruct((N_TOK, 1), jnp.int32),
        scratch_shapes=[
            pltpu.VMEM((N_VOCAB, D), jnp.bfloat16),
        ],
    )(flat, vocab)
    return ids[:, 0]


def _gather_body(ids_hbm, vocab_hbm, out_hbm, idx_v, rows_v, sem):
    wid = lax.axis_index("s") * _NC + lax.axis_index("c")
    base = wid * _BPW
    pltpu.sync_copy(ids_hbm.at[pl.ds(base, _BPW)], idx_v)
    pltpu.async_copy(vocab_hbm.at[idx_v], rows_v, sem).wait()
    pltpu.sync_copy(rows_v, out_hbm.at[pl.ds(base, _BPW)])


@functools.cache
def _sc_gather_kernel():
    return pl.kernel(
        _gather_body,
        out_type=jax.ShapeDtypeStruct((N_TOK, 128), jnp.float32),
        mesh=plsc.VectorSubcoreMesh(core_axis_name="c", subcore_axis_name="s"),
        scratch_types=[
            pltpu.VMEM((_BPW,), jnp.int32),
            pltpu.VMEM((_BPW, 128), jnp.float32),
            pltpu.SemaphoreType.DMA,
        ],
    )


def _clip_body(flat_ref, anc_ref, out_ref):
    f = flat_ref[...]
    a = anc_ref[:, :D]
    off = f - a
    on2 = jnp.sum(off * off, axis=1, keepdims=True)
    an2 = jnp.sum(a * a, axis=1, keepdims=True)
    scale = jnp.minimum(EPS * jnp.sqrt(an2) / (jnp.sqrt(on2) + 1e-8), 1.0)
    out_ref[...] = a + off * scale


def _clip(flat, anchors):
    c_tile = 2048
    return pl.pallas_call(
        _clip_body,
        grid=(N_TOK // c_tile,),
        in_specs=[
            pl.BlockSpec((c_tile, D), lambda t: (t, 0)),
            pl.BlockSpec((c_tile, 128), lambda t: (t, 0)),
        ],
        out_specs=pl.BlockSpec((c_tile, D), lambda t: (t, 0)),
        out_shape=jax.ShapeDtypeStruct((N_TOK, D), jnp.float32),
    )(flat, anchors)


def kernel(embeddings, vocab_embeddings):
    B, S, _ = embeddings.shape
    flat = embeddings.reshape(-1, D)
    ids = _anchor_ids(flat, vocab_embeddings)
    vocab_pad = jnp.pad(vocab_embeddings, ((0, 0), (0, 128 - D)))
    anchors = _sc_gather_kernel()(ids, vocab_pad)
    res = _clip(flat, anchors)
    return res.reshape(B, S, D), ids.reshape(B, S)
